# baseline (speedup 1.0000x reference)
import functools

import jax
import jax.numpy as jnp
from jax import lax
from jax.experimental import pallas as pl
from jax.experimental.pallas import tpu as pltpu
from jax.experimental.pallas import tpu_sc as plsc

T = 8192
D = 4096
E = 64
K = 8
LANES = 16

TB = 512
SB = 1024
NCHUNK = 2


def _matmul_body(x_ref, w_ref, b_ref, o_ref):
    acc = lax.dot_general(
        x_ref[...], w_ref[...],
        dimension_numbers=(((1,), (1,)), ((), ())),
        preferred_element_type=jnp.float32,
    )
    o_ref[...] = acc + b_ref[...]


def _gate_matmul(inp, W, b2d, base, ct):
    nb = base // TB
    return pl.pallas_call(
        _matmul_body,
        grid=(ct // TB,),
        in_specs=[
            pl.BlockSpec((TB, D), lambda i: (i + nb, 0)),
            pl.BlockSpec((E, D), lambda i: (0, 0)),
            pl.BlockSpec((1, E), lambda i: (0, 0)),
        ],
        out_specs=pl.BlockSpec((TB, E), lambda i: (i, 0)),
        out_shape=jax.ShapeDtypeStruct((ct, E), jnp.float32),
    )(inp, W, b2d)


def _merge_top8(ak, av, bk, bv, lane_lt8):
    mk = jnp.where(lane_lt8, ak, lax.rev(bk, (0,)))
    mv = jnp.where(lane_lt8, av, lax.rev(bv, (0,)))
    return plsc.sort_key_val(mk, mv, descending=True)


def _topk_sc(gate):
    info = plsc.get_sparse_core_info()
    NC, NS = info.num_cores, info.num_subcores
    NW = NC * NS
    ct = gate.shape[0]
    RPW = ct // NW

    mesh = plsc.VectorSubcoreMesh(core_axis_name="c", subcore_axis_name="s")

    @functools.partial(
        pl.kernel,
        out_type=jax.ShapeDtypeStruct((ct, LANES), jnp.float32),
        mesh=mesh,
        scratch_types=[
            pltpu.VMEM((RPW, E), jnp.float32),
            pltpu.VMEM((RPW, LANES), jnp.float32),
        ],
        compiler_params=pltpu.CompilerParams(needs_layout_passes=False),
    )
    def k(gate_hbm, out_hbm, g_v, o_v):
        wid = lax.axis_index("s") * NC + lax.axis_index("c")
        base = wid * RPW
        pltpu.sync_copy(gate_hbm.at[pl.ds(base, RPW)], g_v)

        lane = lax.iota(jnp.int32, LANES)
        lane_lt8 = lane < K

        @plsc.parallel_loop(0, RPW, unroll=2)
        def row_body(r):
            sk = []
            sv = []
            for c in range(E // LANES):
                g = g_v[r, pl.ds(c * LANES, LANES)]
                k_, v_ = plsc.sort_key_val(g, lane + c * LANES, descending=True)
                sk.append(k_)
                sv.append(v_)
            k01, v01 = _merge_top8(sk[0], sv[0], sk[1], sv[1], lane_lt8)
            k23, v23 = _merge_top8(sk[2], sv[2], sk[3], sv[3], lane_lt8)
            fk, fv = _merge_top8(k01, v01, k23, v23, lane_lt8)

            m = jnp.max(fk)
            e = jnp.where(lane_lt8, jnp.exp(fk - m), 0.0)
            s = jnp.broadcast_to(jnp.sum(e), (LANES,))
            probs = e / s
            o_v[r, :] = jnp.where(
                lane_lt8, probs, lax.rev(plsc.bitcast(fv, jnp.float32), (0,))
            )

        pltpu.sync_copy(o_v, out_hbm.at[pl.ds(base, RPW)])

    return k(gate)


def _scatter_body(*refs):
    vi_refs = refs[:NCHUNK]
    o_ref = refs[NCHUNK]
    cid = pl.program_id(0)
    vi = jnp.where((cid == 0)[None, None], vi_refs[0][...], vi_refs[-1][...])
    for c in range(1, NCHUNK - 1):
        vi = jnp.where((cid == c)[None, None], vi_refs[c][...], vi)
    expert = lax.broadcasted_iota(jnp.int32, (SB, E), 1)
    acc = jnp.zeros((SB, E), jnp.float32)
    for k in range(K):
        idx_k = lax.bitcast_convert_type(vi[:, 15 - k:16 - k], jnp.int32)
        val_k = vi[:, k:k + 1]
        acc = acc + jnp.where(expert == idx_k, val_k, 0.0)
    o_ref[...] = acc


def _scatter_tc(vis, ct):
    grid = (NCHUNK, ct // SB)
    in_specs = [
        pl.BlockSpec((SB, LANES), lambda c, j: (j, 0)) for _ in range(NCHUNK)
    ]
    return pl.pallas_call(
        _scatter_body,
        grid=grid,
        in_specs=in_specs,
        out_specs=pl.BlockSpec((SB, E), lambda c, j: (c * (ct // SB) + j, 0)),
        out_shape=jax.ShapeDtypeStruct((T, E), jnp.float32),
    )(*vis)


@jax.jit
def kernel(inp, W, b):
    b2d = b.reshape(1, E)
    ct = T // NCHUNK
    vis = []
    for i in range(NCHUNK):
        gate = _gate_matmul(inp, W, b2d, i * ct, ct)
        vis.append(_topk_sc(gate))
    return _scatter_tc(vis, ct)


# speedup vs baseline: 1.2763x; 1.2763x over previous
import functools

import jax
import jax.numpy as jnp
from jax import lax
from jax.experimental import pallas as pl
from jax.experimental.pallas import tpu as pltpu
from jax.experimental.pallas import tpu_sc as plsc

T = 8192
D = 4096
E = 64
K = 8
LANES = 16

TB = 512
CHUNKS = (6656, 1536)


def _matmul_body(x_ref, w_ref, b_ref, o_ref):
    acc = lax.dot_general(
        x_ref[...], w_ref[...],
        dimension_numbers=(((1,), (1,)), ((), ())),
        preferred_element_type=jnp.float32,
    )
    o_ref[...] = acc + b_ref[...]


def _gate_matmul(inp, W, b2d, base, ct):
    nb = base // TB
    return pl.pallas_call(
        _matmul_body,
        grid=(ct // TB,),
        in_specs=[
            pl.BlockSpec((TB, D), lambda i: (i + nb, 0)),
            pl.BlockSpec((E, D), lambda i: (0, 0)),
            pl.BlockSpec((1, E), lambda i: (0, 0)),
        ],
        out_specs=pl.BlockSpec((TB, E), lambda i: (i, 0)),
        out_shape=jax.ShapeDtypeStruct((ct, E), jnp.float32),
    )(inp, W, b2d)


def _merge_top8(ak, av, bk, bv, lane_lt8):
    mk = jnp.where(lane_lt8, ak, lax.rev(bk, (0,)))
    mv = jnp.where(lane_lt8, av, lax.rev(bv, (0,)))
    return plsc.sort_key_val(mk, mv, descending=True)


def _topk_sc(gate, parts=None):
    info = plsc.get_sparse_core_info()
    NC, NS = info.num_cores, info.num_subcores
    NW = NC * NS
    ct = gate.shape[0]
    RPW = ct // NW
    npart = 0 if parts is None else len(parts)
    if parts is None:
        out_rows = ct
        my_off = 0
        part_arrays = ()
        part_offs = ()
        part_rpw = ()
    else:
        part_arrays = tuple(p for p, _ in parts)
        part_offs = tuple(o for _, o in parts)
        part_rpw = tuple(p.shape[0] // NW for p, _ in parts)
        out_rows = T
        my_off = sum(p.shape[0] for p, _ in parts)

    mesh = plsc.VectorSubcoreMesh(core_axis_name="c", subcore_axis_name="s")

    @functools.partial(
        pl.kernel,
        out_type=jax.ShapeDtypeStruct((out_rows, E), jnp.float32),
        mesh=mesh,
        scratch_types=[
            pltpu.VMEM((RPW, E), jnp.float32),
            pltpu.VMEM((RPW, E), jnp.float32),
        ]
        + [pltpu.VMEM((r, E), jnp.float32) for r in part_rpw]
        + [pltpu.SemaphoreType.DMA for _ in range(2 * npart)],
        compiler_params=pltpu.CompilerParams(needs_layout_passes=False),
    )
    def k(gate_hbm, *rest):
        part_hbm = rest[:npart]
        out_hbm, g_v, o_v = rest[npart:npart + 3]
        c_v = rest[npart + 3:npart + 3 + npart]
        sems = rest[npart + 3 + npart:]
        wid = lax.axis_index("s") * NC + lax.axis_index("c")
        base = wid * RPW
        ins = [
            pltpu.async_copy(
                part_hbm[i].at[pl.ds(wid * part_rpw[i], part_rpw[i])],
                c_v[i], sems[i],
            )
            for i in range(npart)
        ]
        pltpu.sync_copy(gate_hbm.at[pl.ds(base, RPW)], g_v)

        lane = lax.iota(jnp.int32, LANES)
        lane_lt8 = lane < K
        zeros16 = jnp.zeros((LANES,), jnp.float32)

        @plsc.parallel_loop(0, RPW, unroll=2)
        def row_body(r):
            sk = []
            sv = []
            for c in range(E // LANES):
                g = g_v[r, pl.ds(c * LANES, LANES)]
                k_, v_ = plsc.sort_key_val(g, lane + c * LANES, descending=True)
                sk.append(k_)
                sv.append(v_)
            k01, v01 = _merge_top8(sk[0], sv[0], sk[1], sv[1], lane_lt8)
            k23, v23 = _merge_top8(sk[2], sv[2], sk[3], sv[3], lane_lt8)
            fk, fv = _merge_top8(k01, v01, k23, v23, lane_lt8)

            m = jnp.max(fk)
            e = jnp.where(lane_lt8, jnp.exp(fk - m), 0.0)
            s = jnp.broadcast_to(jnp.sum(e), (LANES,))
            probs = e / s

            for c in range(E // LANES):
                o_v[r, pl.ds(c * LANES, LANES)] = zeros16
            rows = jnp.full((LANES,), r, jnp.int32)
            plsc.store_scatter(o_v, [rows, fv], probs, mask=lane_lt8)

        outs = []
        for i in range(npart):
            ins[i].wait()
            outs.append(
                pltpu.async_copy(
                    c_v[i],
                    out_hbm.at[pl.ds(part_offs[i] + wid * part_rpw[i],
                                     part_rpw[i])],
                    sems[npart + i],
                )
            )
        pltpu.sync_copy(o_v, out_hbm.at[pl.ds(my_off + base, RPW)])
        for o in outs:
            o.wait()

    if parts is None:
        return k(gate)
    return k(gate, *part_arrays)


@jax.jit
def kernel(inp, W, b):
    b2d = b.reshape(1, E)
    parts = []
    off = 0
    for ct in CHUNKS[:-1]:
        gate = _gate_matmul(inp, W, b2d, off, ct)
        parts.append((_topk_sc(gate), off))
        off += ct
    gate = _gate_matmul(inp, W, b2d, off, CHUNKS[-1])
    return _topk_sc(gate, parts=parts)
